# baseline (speedup 1.0000x reference)
import jax
import jax.numpy as jnp
from jax import lax
from jax.experimental import pallas as pl
from jax.experimental.pallas import tpu as pltpu
from jax.experimental.pallas import tpu_sc as plsc

N = 10000
E = 320000
D = 128
D2 = 2 * D
DE = 16
H = 8
C = 16
SCALE = 0.25

NC = 2
NS = 16
NW = NC * NS
EPW = E // NW
B = 80
NCH = EPW // B
N_PAD = 10240
RPT = N_PAD // NS


def _proj_body(x_ref, w_ref, b_ref, o_ref):
    o_ref[...] = jnp.dot(x_ref[...], w_ref[...],
                         preferred_element_type=jnp.float32) + b_ref[...]


def _proj(x, w, b2d):
    bn = 2000
    grid = N // bn
    row = pl.BlockSpec((bn, D), lambda i: (i, 0))
    return pl.pallas_call(
        _proj_body,
        grid=(grid,),
        in_specs=[row, pl.BlockSpec((D, D), lambda i: (0, 0)),
                  pl.BlockSpec((1, D), lambda i: (0, 0))],
        out_specs=row,
        out_shape=jax.ShapeDtypeStruct((N, D), jnp.float32),
    )(x, w, b2d)


def _qkvs_body(h_ref, wq_ref, wk_ref, wv_ref, ws_ref, bq_ref, bk_ref, bv_ref,
               bs_ref, q_ref, kv_ref, s_ref):
    hb = h_ref[...]
    q_ref[...] = jnp.dot(hb, wq_ref[...], preferred_element_type=jnp.float32) + bq_ref[...]
    k = jnp.dot(hb, wk_ref[...], preferred_element_type=jnp.float32) + bk_ref[...]
    v = jnp.dot(hb, wv_ref[...], preferred_element_type=jnp.float32) + bv_ref[...]
    kv_ref[...] = jnp.concatenate([k, v], axis=1)
    s_ref[...] = jnp.dot(hb, ws_ref[...], preferred_element_type=jnp.float32) + bs_ref[...]


def _qkvs(h, wq, wk, wv, ws, bq, bk, bv, bs):
    bn = 2000
    grid = N // bn
    row = pl.BlockSpec((bn, D), lambda i: (i, 0))
    mat = pl.BlockSpec((D, D), lambda i: (0, 0))
    bias = pl.BlockSpec((1, D), lambda i: (0, 0))
    return pl.pallas_call(
        _qkvs_body,
        grid=(grid,),
        in_specs=[row, mat, mat, mat, mat, bias, bias, bias, bias],
        out_specs=[row, pl.BlockSpec((bn, D2), lambda i: (i, 0)), row],
        out_shape=[jax.ShapeDtypeStruct((N, D), jnp.float32),
                   jax.ShapeDtypeStruct((N, D2), jnp.float32),
                   jax.ShapeDtypeStruct((N, D), jnp.float32)],
    )(h, wq, wk, wv, ws, bq, bk, bv, bs)


def _ee_body(ea_ref, ew_ref, we_ref, eb_ref, ee_ref):
    w2 = jnp.dot(ew_ref[...], we_ref[...], preferred_element_type=jnp.float32)
    b2 = jnp.dot(eb_ref[...], we_ref[...], preferred_element_type=jnp.float32)
    ee_ref[...] = jnp.dot(ea_ref[...], w2, preferred_element_type=jnp.float32) + b2


def _ee(edge_attr, edge_W, edge_b2, We_l):
    be = 4000
    grid = E // be
    return pl.pallas_call(
        _ee_body,
        grid=(grid,),
        in_specs=[pl.BlockSpec((be, DE), lambda i: (i, 0)),
                  pl.BlockSpec((DE, D), lambda i: (0, 0)),
                  pl.BlockSpec((D, D), lambda i: (0, 0)),
                  pl.BlockSpec((1, D), lambda i: (0, 0))],
        out_specs=pl.BlockSpec((be, D), lambda i: (i, 0)),
        out_shape=jax.ShapeDtypeStruct((E, D), jnp.float32),
    )(edge_attr, edge_W, We_l, edge_b2)


def _norm_body(a0_ref, a1_ref, d0_ref, d1_ref, skip_ref, h_ref, o_ref):
    denb = d0_ref[...] + d1_ref[...]
    agg = (a0_ref[...] + a1_ref[...]) / (denb + 1e-16) + skip_ref[...]
    o_ref[...] = h_ref[...] + jnp.maximum(agg, 0.0)


def _norm_update(acc0, acc1, den0, den1, skip, h):
    bn = 2000
    grid = N // bn
    row = pl.BlockSpec((bn, D), lambda i: (i, 0))
    return pl.pallas_call(
        _norm_body,
        grid=(grid,),
        in_specs=[row, row, row, row, row, row],
        out_specs=row,
        out_shape=jax.ShapeDtypeStruct((N, D), jnp.float32),
    )(acc0, acc1, den0, den1, skip, h)


def _final_body(h_ref, g_ref, b_ref, ow_ref, ob_ref, o_ref, acc_ref):
    i = pl.program_id(0)
    nblk = pl.num_programs(0)
    hb = h_ref[...]
    mu = jnp.mean(hb, axis=1, keepdims=True)
    var = jnp.mean((hb - mu) ** 2, axis=1, keepdims=True)
    hn = (hb - mu) / jnp.sqrt(var + 1e-5) * g_ref[...] + b_ref[...]
    psum = jnp.sum(hn, axis=0, keepdims=True)

    @pl.when(i == 0)
    def _():
        acc_ref[...] = jnp.zeros_like(acc_ref)

    acc_ref[...] += psum

    @pl.when(i == nblk - 1)
    def _():
        o_ref[...] = jnp.dot(acc_ref[...] * (1.0 / N), ow_ref[...],
                             preferred_element_type=jnp.float32) + ob_ref[...]


def _final(h, ln_g2d, ln_b2d, out_W, out_b2d):
    bn = 2000
    grid = N // bn
    return pl.pallas_call(
        _final_body,
        grid=(grid,),
        in_specs=[pl.BlockSpec((bn, D), lambda i: (i, 0)),
                  pl.BlockSpec((1, D), lambda i: (0, 0)),
                  pl.BlockSpec((1, D), lambda i: (0, 0)),
                  pl.BlockSpec((D, D), lambda i: (0, 0)),
                  pl.BlockSpec((1, D), lambda i: (0, 0))],
        out_specs=pl.BlockSpec((1, D), lambda i: (0, 0)),
        out_shape=jax.ShapeDtypeStruct((1, D), jnp.float32),
        scratch_shapes=[pltpu.VMEM((1, D), jnp.float32)],
    )(h, ln_g2d, ln_b2d, out_W, out_b2d)


BF = 40
NCHF = EPW // BF


def _sc_fused_body(kvT, qT, ee, src, dst, znode, acc_out, w8_out,
                   tab_sh, idxs, idxd, kvb, qb, eeb, msgb, wb,
                   sg0, sg1, sw0):
    cid = lax.axis_index("c")
    sid = lax.axis_index("s")
    wid = cid * NS + sid
    base = wid * EPW
    r0 = sid * RPT
    off = cid * N_PAD + r0
    sg = (sg0, sg1)
    i15 = jnp.full((C,), C - 1, jnp.int32)

    pltpu.sync_copy(znode.at[pl.ds(r0, RPT)], tab_sh.at[pl.ds(r0, RPT)])
    plsc.subcore_barrier()

    def issue(c, p):
        eb = base + c * BF
        pltpu.sync_copy(src.at[pl.ds(eb, BF)], idxs.at[p])
        pltpu.sync_copy(dst.at[pl.ds(eb, BF)], idxd.at[p])
        pltpu.async_copy(kvT.at[idxs.at[p]], kvb.at[p], sg[p])
        pltpu.async_copy(qT.at[idxd.at[p]], qb.at[p], sg[p])

    def wait_in(p):
        pltpu.make_async_copy(kvT.at[idxs.at[p]], kvb.at[p], sg[p]).wait()
        pltpu.make_async_copy(qT.at[idxd.at[p]], qb.at[p], sg[p]).wait()

    def wait_wout():
        pltpu.make_async_copy(wb, w8_out.at[pl.ds(base, BF)], sw0).wait()

    def compute_scatter(c, p):
        eb = base + c * BF
        pltpu.sync_copy(ee.at[pl.ds(eb, BF)], eeb)

        @plsc.parallel_loop(0, BF, 1, unroll=8)
        def _(e):
            for h in range(H):
                sl_h = pl.ds(h * C, C)
                qh = qb[p, e, sl_h]
                kh = kvb[p, e, sl_h]
                vh = kvb[p, e, pl.ds(D + h * C, C)]
                eh = eeb[e, sl_h]
                vpe = vh + eh
                t = qh * (kh + eh)
                cs = plsc.cumsum(t * SCALE)
                s16 = cs.at[i15].get(mode="promise_in_bounds")
                exv = jnp.exp(s16)
                msgb[e, sl_h] = exv * vpe
                wb[e, sl_h] = exv

        pltpu.sync_copy(msgb, tab_sh.at[idxd.at[p]], add=True)
        pltpu.async_copy(wb, w8_out.at[pl.ds(eb, BF)], sw0)

    issue(0, 0)

    @pl.loop(0, NCHF, step=2)
    def _(c):
        issue(c + 1, 1)
        wait_in(0)

        @pl.when(c > 0)
        def _():
            wait_wout()

        compute_scatter(c, 0)

        @pl.when(c + 2 < NCHF)
        def _():
            issue(c + 2, 0)

        wait_in(1)
        wait_wout()
        compute_scatter(c + 1, 1)

    wait_wout()
    plsc.subcore_barrier()
    pltpu.sync_copy(tab_sh.at[pl.ds(r0, RPT)], acc_out.at[pl.ds(off, RPT)])


def _sc_fused(kvT, qT, ee, src, dst, znode):
    mesh = plsc.VectorSubcoreMesh(core_axis_name="c", subcore_axis_name="s")
    f = pl.kernel(
        _sc_fused_body,
        out_type=[jax.ShapeDtypeStruct((NC * N_PAD, D), jnp.float32),
                  jax.ShapeDtypeStruct((E, D), jnp.float32)],
        mesh=mesh,
        compiler_params=pltpu.CompilerParams(needs_layout_passes=False),
        scratch_types=[
            pltpu.VMEM_SHARED((N_PAD, D), jnp.float32),
            pltpu.VMEM((2, BF), jnp.int32),
            pltpu.VMEM((2, BF), jnp.int32),
            pltpu.VMEM((2, BF, D2), jnp.float32),
            pltpu.VMEM((2, BF, D), jnp.float32),
            pltpu.VMEM((BF, D), jnp.float32),
            pltpu.VMEM((BF, D), jnp.float32),
            pltpu.VMEM((BF, D), jnp.float32),
            pltpu.SemaphoreType.DMA,
            pltpu.SemaphoreType.DMA,
            pltpu.SemaphoreType.DMA,
        ],
    )
    accP, w8 = f(kvT, qT, ee, src, dst, znode)
    return accP.reshape(NC, N_PAD, D), w8


def _sc_scatter_body(w8, dst, znode, den_out,
                     tab_sh, idx, mb, sl0, sl1):
    cid = lax.axis_index("c")
    sid = lax.axis_index("s")
    wid = cid * NS + sid
    base = wid * EPW
    r0 = sid * RPT
    off = cid * N_PAD + r0
    sl = (sl0, sl1)

    def accumulate(src_arr, out_arr):
        pltpu.sync_copy(znode.at[pl.ds(r0, RPT)], tab_sh.at[pl.ds(r0, RPT)])
        plsc.subcore_barrier()

        def issue_load(c, p):
            eb = base + c * B
            pltpu.async_copy(dst.at[pl.ds(eb, B)], idx.at[p], sl[p])
            pltpu.async_copy(src_arr.at[pl.ds(eb, B)], mb.at[p], sl[p])

        def wait_load(p):
            pltpu.make_async_copy(dst.at[pl.ds(base, B)], idx.at[p], sl[p]).wait()
            pltpu.make_async_copy(src_arr.at[pl.ds(base, B)], mb.at[p], sl[p]).wait()

        def scatter(p):
            pltpu.sync_copy(mb.at[p], tab_sh.at[idx.at[p]], add=True)

        issue_load(0, 0)

        @pl.loop(0, NCH - 1, step=2)
        def _(c):
            issue_load(c + 1, 1)
            wait_load(0)
            scatter(0)
            issue_load(c + 2, 0)
            wait_load(1)
            scatter(1)

        wait_load(0)
        scatter(0)
        plsc.subcore_barrier()
        pltpu.sync_copy(tab_sh.at[pl.ds(r0, RPT)], out_arr.at[pl.ds(off, RPT)])
        plsc.subcore_barrier()

    accumulate(w8, den_out)


def _sc_scatter(w8, dst, znode):
    mesh = plsc.VectorSubcoreMesh(core_axis_name="c", subcore_axis_name="s")
    f = pl.kernel(
        _sc_scatter_body,
        out_type=jax.ShapeDtypeStruct((NC * N_PAD, D), jnp.float32),
        mesh=mesh,
        scratch_types=[
            pltpu.VMEM_SHARED((N_PAD, D), jnp.float32),
            pltpu.VMEM((2, B), jnp.int32),
            pltpu.VMEM((2, B, D), jnp.float32),
            pltpu.SemaphoreType.DMA,
            pltpu.SemaphoreType.DMA,
        ],
    )
    denP = f(w8, dst, znode)
    return denP.reshape(NC, N_PAD, D)


def kernel(x, edge_index, edge_attr, node_W, node_b, edge_W, edge_b, Wq, bq,
           Wk, bk, Wv, bv, We, Wskip, bskip, ln_g, ln_b, out_W, out_b):
    src = edge_index[0]
    dst = edge_index[1]

    znode = jnp.zeros((N_PAD, D), jnp.float32)

    node_b2 = node_b.reshape(1, D)
    edge_b2 = edge_b.reshape(1, D)

    h = _proj(x, node_W, node_b2)

    for l in range(2):
        q, kv, skip = _qkvs(h, Wq[l], Wk[l], Wv[l], Wskip[l],
                            bq[l].reshape(1, D), bk[l].reshape(1, D),
                            bv[l].reshape(1, D), bskip[l].reshape(1, D))
        ee = _ee(edge_attr, edge_W, edge_b2, We[l])
        accP, w8 = _sc_fused(kv, q, ee, src, dst, znode)
        denP = _sc_scatter(w8, dst, znode)
        h = _norm_update(accP[0, :N], accP[1, :N], denP[0, :N], denP[1, :N],
                         skip, h)

    return _final(h, ln_g.reshape(1, D), ln_b.reshape(1, D), out_W,
                  out_b.reshape(1, D))


# speedup vs baseline: 2.2428x; 2.2428x over previous
import jax
import jax.numpy as jnp
from jax import lax
from jax.experimental import pallas as pl
from jax.experimental.pallas import tpu as pltpu
from jax.experimental.pallas import tpu_sc as plsc

N = 10000
E = 320000
D = 128
D2 = 2 * D
DE = 16
H = 8
C = 16
SCALE = 0.25

NC = 2
NS = 16
NW = NC * NS
EH = E // 2
EPW = EH // NW
B = 40
NCH = EPW // B
N_PAD = 10240
RPT = N_PAD // NS


def _proj_body(x_ref, w_ref, b_ref, o_ref):
    o_ref[...] = jnp.dot(x_ref[...], w_ref[...],
                         preferred_element_type=jnp.float32) + b_ref[...]


def _proj(x, w, b2d):
    bn = 2000
    grid = N // bn
    row = pl.BlockSpec((bn, D), lambda i: (i, 0))
    return pl.pallas_call(
        _proj_body,
        grid=(grid,),
        in_specs=[row, pl.BlockSpec((D, D), lambda i: (0, 0)),
                  pl.BlockSpec((1, D), lambda i: (0, 0))],
        out_specs=row,
        out_shape=jax.ShapeDtypeStruct((N, D), jnp.float32),
    )(x, w, b2d)


def _qkvs_body(h_ref, wq_ref, wk_ref, wv_ref, ws_ref, bq_ref, bk_ref, bv_ref,
               bs_ref, q_ref, kv_ref, s_ref):
    hb = h_ref[...]
    q_ref[...] = jnp.dot(hb, wq_ref[...], preferred_element_type=jnp.float32) + bq_ref[...]
    k = jnp.dot(hb, wk_ref[...], preferred_element_type=jnp.float32) + bk_ref[...]
    v = jnp.dot(hb, wv_ref[...], preferred_element_type=jnp.float32) + bv_ref[...]
    kv_ref[...] = jnp.concatenate([k, v], axis=1)
    s_ref[...] = jnp.dot(hb, ws_ref[...], preferred_element_type=jnp.float32) + bs_ref[...]


def _qkvs(h, wq, wk, wv, ws, bq, bk, bv, bs):
    bn = 2000
    grid = N // bn
    row = pl.BlockSpec((bn, D), lambda i: (i, 0))
    mat = pl.BlockSpec((D, D), lambda i: (0, 0))
    bias = pl.BlockSpec((1, D), lambda i: (0, 0))
    return pl.pallas_call(
        _qkvs_body,
        grid=(grid,),
        in_specs=[row, mat, mat, mat, mat, bias, bias, bias, bias],
        out_specs=[row, pl.BlockSpec((bn, D2), lambda i: (i, 0)), row],
        out_shape=[jax.ShapeDtypeStruct((N, D), jnp.float32),
                   jax.ShapeDtypeStruct((N, D2), jnp.float32),
                   jax.ShapeDtypeStruct((N, D), jnp.float32)],
    )(h, wq, wk, wv, ws, bq, bk, bv, bs)


def _edge_body(qi_ref, kvj_ref, ea_ref, ew_ref, we_ref, eb_ref, g_ref, s_ref,
               m_ref, w_ref):
    w2 = jnp.dot(ew_ref[...], we_ref[...], preferred_element_type=jnp.float32)
    b2 = jnp.dot(eb_ref[...], we_ref[...], preferred_element_type=jnp.float32)
    eeb = jnp.dot(ea_ref[...], w2, preferred_element_type=jnp.float32) + b2
    kvj = kvj_ref[...]
    kj = kvj[:, :D] + eeb
    vj = kvj[:, D:] + eeb
    t = qi_ref[...] * kj
    alpha = jnp.dot(t, g_ref[...], preferred_element_type=jnp.float32)
    ex = jnp.exp(alpha * SCALE)
    exb = jnp.dot(ex, s_ref[...], preferred_element_type=jnp.float32)
    w_ref[...] = exb
    m_ref[...] = exb * vj


def _edge_compute(qi, kvj, edge_attr, edge_W, We_l, edge_b2, gsel, ssel, e0):
    be = 2000
    grid = EH // be
    hoff = e0 // be
    row = pl.BlockSpec((be, D), lambda i: (i, 0))
    return pl.pallas_call(
        _edge_body,
        grid=(grid,),
        in_specs=[row,
                  pl.BlockSpec((be, D2), lambda i: (i, 0)),
                  pl.BlockSpec((be, DE), lambda i: (i + hoff, 0)),
                  pl.BlockSpec((DE, D), lambda i: (0, 0)),
                  pl.BlockSpec((D, D), lambda i: (0, 0)),
                  pl.BlockSpec((1, D), lambda i: (0, 0)),
                  pl.BlockSpec((D, H), lambda i: (0, 0)),
                  pl.BlockSpec((H, D), lambda i: (0, 0))],
        out_specs=[row, row],
        out_shape=[jax.ShapeDtypeStruct((EH, D), jnp.float32),
                   jax.ShapeDtypeStruct((EH, D), jnp.float32)],
    )(qi, kvj, edge_attr, edge_W, We_l, edge_b2, gsel, ssel)


def _norm_body(a0_ref, a1_ref, a2_ref, a3_ref, d0_ref, d1_ref, d2_ref, d3_ref,
               skip_ref, h_ref, o_ref):
    denb = d0_ref[...] + d1_ref[...] + d2_ref[...] + d3_ref[...]
    accb = a0_ref[...] + a1_ref[...] + a2_ref[...] + a3_ref[...]
    agg = accb / (denb + 1e-16) + skip_ref[...]
    o_ref[...] = h_ref[...] + jnp.maximum(agg, 0.0)


def _norm_update(accs, dens, skip, h):
    bn = 2000
    grid = N // bn
    row = pl.BlockSpec((bn, D), lambda i: (i, 0))
    return pl.pallas_call(
        _norm_body,
        grid=(grid,),
        in_specs=[row] * 10,
        out_specs=row,
        out_shape=jax.ShapeDtypeStruct((N, D), jnp.float32),
    )(*accs, *dens, skip, h)


def _final_body(h_ref, g_ref, b_ref, ow_ref, ob_ref, o_ref, acc_ref):
    i = pl.program_id(0)
    nblk = pl.num_programs(0)
    hb = h_ref[...]
    mu = jnp.mean(hb, axis=1, keepdims=True)
    var = jnp.mean((hb - mu) ** 2, axis=1, keepdims=True)
    hn = (hb - mu) / jnp.sqrt(var + 1e-5) * g_ref[...] + b_ref[...]
    psum = jnp.sum(hn, axis=0, keepdims=True)

    @pl.when(i == 0)
    def _():
        acc_ref[...] = jnp.zeros_like(acc_ref)

    acc_ref[...] += psum

    @pl.when(i == nblk - 1)
    def _():
        o_ref[...] = jnp.dot(acc_ref[...] * (1.0 / N), ow_ref[...],
                             preferred_element_type=jnp.float32) + ob_ref[...]


def _final(h, ln_g2d, ln_b2d, out_W, out_b2d):
    bn = 2000
    grid = N // bn
    return pl.pallas_call(
        _final_body,
        grid=(grid,),
        in_specs=[pl.BlockSpec((bn, D), lambda i: (i, 0)),
                  pl.BlockSpec((1, D), lambda i: (0, 0)),
                  pl.BlockSpec((1, D), lambda i: (0, 0)),
                  pl.BlockSpec((D, D), lambda i: (0, 0)),
                  pl.BlockSpec((1, D), lambda i: (0, 0))],
        out_specs=pl.BlockSpec((1, D), lambda i: (0, 0)),
        out_shape=jax.ShapeDtypeStruct((1, D), jnp.float32),
        scratch_shapes=[pltpu.VMEM((1, D), jnp.float32)],
    )(h, ln_g2d, ln_b2d, out_W, out_b2d)


def _sc_gather_body(kvT, qT, src, dst, kvj_out, qi_out,
                    idxs, idxd, kvb, qb, sg0, sg1, sw0, sw1, *, e0):
    cid = lax.axis_index("c")
    sid = lax.axis_index("s")
    wid = cid * NS + sid
    base = wid * EPW
    sg = (sg0, sg1)
    sw = (sw0, sw1)

    def issue_gather(c, p):
        eb = base + c * B
        pltpu.sync_copy(src.at[pl.ds(e0 + eb, B)], idxs.at[p])
        pltpu.sync_copy(dst.at[pl.ds(e0 + eb, B)], idxd.at[p])
        pltpu.async_copy(kvT.at[idxs.at[p]], kvb.at[p], sg[p])
        pltpu.async_copy(qT.at[idxd.at[p]], qb.at[p], sg[p])

    def wait_gather(p):
        pltpu.make_async_copy(kvT.at[idxs.at[p]], kvb.at[p], sg[p]).wait()
        pltpu.make_async_copy(qT.at[idxd.at[p]], qb.at[p], sg[p]).wait()

    def issue_write(c, p):
        eb = base + c * B
        pltpu.async_copy(kvb.at[p], kvj_out.at[pl.ds(eb, B)], sw[p])
        pltpu.async_copy(qb.at[p], qi_out.at[pl.ds(eb, B)], sw[p])

    def wait_write(p):
        pltpu.make_async_copy(kvb.at[p], kvj_out.at[pl.ds(base, B)], sw[p]).wait()
        pltpu.make_async_copy(qb.at[p], qi_out.at[pl.ds(base, B)], sw[p]).wait()

    issue_gather(0, 0)

    @pl.loop(0, NCH - 1, step=2)
    def _(c):
        @pl.when(c > 0)
        def _():
            wait_write(1)

        issue_gather(c + 1, 1)
        wait_gather(0)
        issue_write(c, 0)
        wait_gather(1)
        issue_write(c + 1, 1)
        wait_write(0)
        issue_gather(c + 2, 0)

    wait_gather(0)
    wait_write(1)
    issue_write(NCH - 1, 0)
    wait_write(0)


def _sc_gather(kvT, qT, src, dst, e0):
    mesh = plsc.VectorSubcoreMesh(core_axis_name="c", subcore_axis_name="s")
    import functools as _ft
    f = pl.kernel(
        _ft.partial(_sc_gather_body, e0=e0),
        out_type=[jax.ShapeDtypeStruct((EH, D2), jnp.float32),
                  jax.ShapeDtypeStruct((EH, D), jnp.float32)],
        mesh=mesh,
        scratch_types=[
            pltpu.VMEM((2, B), jnp.int32),
            pltpu.VMEM((2, B), jnp.int32),
            pltpu.VMEM((2, B, D2), jnp.float32),
            pltpu.VMEM((2, B, D), jnp.float32),
            pltpu.SemaphoreType.DMA,
            pltpu.SemaphoreType.DMA,
            pltpu.SemaphoreType.DMA,
            pltpu.SemaphoreType.DMA,
        ],
    )
    return f(kvT, qT, src, dst)


def _sc_scatter_body(msg, w8, dst, znode, acc_out, den_out,
                     tab_sh, idx, mb, sl0, sl1, *, e0):
    cid = lax.axis_index("c")
    sid = lax.axis_index("s")
    wid = cid * NS + sid
    base = wid * EPW
    r0 = sid * RPT
    off = cid * N_PAD + r0
    sl = (sl0, sl1)

    def accumulate(src_arr, out_arr):
        pltpu.sync_copy(znode.at[pl.ds(r0, RPT)], tab_sh.at[pl.ds(r0, RPT)])
        plsc.subcore_barrier()

        def issue_load(c, p):
            eb = base + c * B
            pltpu.async_copy(dst.at[pl.ds(e0 + eb, B)], idx.at[p], sl[p])
            pltpu.async_copy(src_arr.at[pl.ds(eb, B)], mb.at[p], sl[p])

        def wait_load(p):
            pltpu.make_async_copy(dst.at[pl.ds(base, B)], idx.at[p], sl[p]).wait()
            pltpu.make_async_copy(src_arr.at[pl.ds(base, B)], mb.at[p], sl[p]).wait()

        def scatter(p):
            pltpu.sync_copy(mb.at[p], tab_sh.at[idx.at[p]], add=True)

        issue_load(0, 0)

        @pl.loop(0, NCH - 1, step=2)
        def _(c):
            issue_load(c + 1, 1)
            wait_load(0)
            scatter(0)
            issue_load(c + 2, 0)
            wait_load(1)
            scatter(1)

        wait_load(0)
        scatter(0)
        plsc.subcore_barrier()
        pltpu.sync_copy(tab_sh.at[pl.ds(r0, RPT)], out_arr.at[pl.ds(off, RPT)])
        plsc.subcore_barrier()

    accumulate(msg, acc_out)
    accumulate(w8, den_out)


def _sc_scatter(msg, w8, dst, znode, e0):
    mesh = plsc.VectorSubcoreMesh(core_axis_name="c", subcore_axis_name="s")
    import functools as _ft
    f = pl.kernel(
        _ft.partial(_sc_scatter_body, e0=e0),
        out_type=[jax.ShapeDtypeStruct((NC * N_PAD, D), jnp.float32),
                  jax.ShapeDtypeStruct((NC * N_PAD, D), jnp.float32)],
        mesh=mesh,
        scratch_types=[
            pltpu.VMEM_SHARED((N_PAD, D), jnp.float32),
            pltpu.VMEM((2, B), jnp.int32),
            pltpu.VMEM((2, B, D), jnp.float32),
            pltpu.SemaphoreType.DMA,
            pltpu.SemaphoreType.DMA,
        ],
    )
    accP, denP = f(msg, w8, dst, znode)
    return accP.reshape(NC, N_PAD, D), denP.reshape(NC, N_PAD, D)


def kernel(x, edge_index, edge_attr, node_W, node_b, edge_W, edge_b, Wq, bq,
           Wk, bk, Wv, bv, We, Wskip, bskip, ln_g, ln_b, out_W, out_b):
    src = edge_index[0]
    dst = edge_index[1]

    lane = jnp.arange(D, dtype=jnp.int32)
    head = jnp.arange(H, dtype=jnp.int32)
    gsel = (lane[:, None] // C == head[None, :]).astype(jnp.float32)
    ssel = gsel.T.copy()

    znode = jnp.zeros((N_PAD, D), jnp.float32)

    node_b2 = node_b.reshape(1, D)
    edge_b2 = edge_b.reshape(1, D)

    h = _proj(x, node_W, node_b2)

    for l in range(2):
        q, kv, skip = _qkvs(h, Wq[l], Wk[l], Wv[l], Wskip[l],
                            bq[l].reshape(1, D), bk[l].reshape(1, D),
                            bv[l].reshape(1, D), bskip[l].reshape(1, D))
        accs = []
        dens = []
        for e0 in (0, EH):
            kvj, qi = _sc_gather(kv, q, src, dst, e0)
            msg, w8 = _edge_compute(qi, kvj, edge_attr, edge_W, We[l],
                                    edge_b2, gsel, ssel, e0)
            accP, denP = _sc_scatter(msg, w8, dst, znode, e0)
            accs += [accP[0, :N], accP[1, :N]]
            dens += [denP[0, :N], denP[1, :N]]
        h = _norm_update(accs, dens, skip, h)

    return _final(h, ln_g.reshape(1, D), ln_b.reshape(1, D), out_W,
                  out_b.reshape(1, D))


# speedup vs baseline: 2.4940x; 1.1120x over previous
import jax
import jax.numpy as jnp
from jax import lax
from jax.experimental import pallas as pl
from jax.experimental.pallas import tpu as pltpu
from jax.experimental.pallas import tpu_sc as plsc

N = 10000
E = 320000
D = 128
D2 = 2 * D
DE = 16
H = 8
C = 16
SCALE = 0.25

NC = 2
NS = 16
NW = NC * NS
EPW = E // NW
B = 80
NCH = EPW // B
N_PAD = 10240
RPT = N_PAD // NS


def _proj_body(x_ref, w_ref, b_ref, o_ref):
    o_ref[...] = jnp.dot(x_ref[...], w_ref[...],
                         preferred_element_type=jnp.float32) + b_ref[...]


def _proj(x, w, b2d):
    bn = 2000
    grid = N // bn
    row = pl.BlockSpec((bn, D), lambda i: (i, 0))
    return pl.pallas_call(
        _proj_body,
        grid=(grid,),
        in_specs=[row, pl.BlockSpec((D, D), lambda i: (0, 0)),
                  pl.BlockSpec((1, D), lambda i: (0, 0))],
        out_specs=row,
        out_shape=jax.ShapeDtypeStruct((N, D), jnp.float32),
    )(x, w, b2d)


def _qkvs_body(h_ref, wq_ref, wk_ref, wv_ref, ws_ref, bq_ref, bk_ref, bv_ref,
               bs_ref, q_ref, kv_ref, s_ref):
    hb = h_ref[...]
    q_ref[...] = jnp.dot(hb, wq_ref[...], preferred_element_type=jnp.float32) + bq_ref[...]
    k = jnp.dot(hb, wk_ref[...], preferred_element_type=jnp.float32) + bk_ref[...]
    v = jnp.dot(hb, wv_ref[...], preferred_element_type=jnp.float32) + bv_ref[...]
    kv_ref[...] = jnp.concatenate([k, v], axis=1)
    s_ref[...] = jnp.dot(hb, ws_ref[...], preferred_element_type=jnp.float32) + bs_ref[...]


def _qkvs(h, wq, wk, wv, ws, bq, bk, bv, bs):
    bn = 2000
    grid = N // bn
    row = pl.BlockSpec((bn, D), lambda i: (i, 0))
    mat = pl.BlockSpec((D, D), lambda i: (0, 0))
    bias = pl.BlockSpec((1, D), lambda i: (0, 0))
    return pl.pallas_call(
        _qkvs_body,
        grid=(grid,),
        in_specs=[row, mat, mat, mat, mat, bias, bias, bias, bias],
        out_specs=[row, pl.BlockSpec((bn, D2), lambda i: (i, 0)), row],
        out_shape=[jax.ShapeDtypeStruct((N, D), jnp.float32),
                   jax.ShapeDtypeStruct((N, D2), jnp.float32),
                   jax.ShapeDtypeStruct((N, D), jnp.float32)],
    )(h, wq, wk, wv, ws, bq, bk, bv, bs)


def _edge_body(qi_ref, kvj_ref, ea_ref, ew_ref, we_ref, eb_ref, g_ref, s_ref,
               m_ref, w_ref):
    w2 = jnp.dot(ew_ref[...], we_ref[...], preferred_element_type=jnp.float32)
    b2 = jnp.dot(eb_ref[...], we_ref[...], preferred_element_type=jnp.float32)
    eeb = jnp.dot(ea_ref[...], w2, preferred_element_type=jnp.float32) + b2
    kvj = kvj_ref[...]
    kj = kvj[:, :D] + eeb
    vj = kvj[:, D:] + eeb
    t = qi_ref[...] * kj
    alpha = jnp.dot(t, g_ref[...], preferred_element_type=jnp.float32)
    ex = jnp.exp(alpha * SCALE)
    exb = jnp.dot(ex, s_ref[...], preferred_element_type=jnp.float32)
    w_ref[...] = exb
    m_ref[...] = exb * vj


def _edge_compute(qi, kvj, edge_attr, edge_W, We_l, edge_b2, gsel, ssel):
    be = 2000
    grid = E // be
    row = pl.BlockSpec((be, D), lambda i: (i, 0))
    return pl.pallas_call(
        _edge_body,
        grid=(grid,),
        in_specs=[row,
                  pl.BlockSpec((be, D2), lambda i: (i, 0)),
                  pl.BlockSpec((be, DE), lambda i: (i, 0)),
                  pl.BlockSpec((DE, D), lambda i: (0, 0)),
                  pl.BlockSpec((D, D), lambda i: (0, 0)),
                  pl.BlockSpec((1, D), lambda i: (0, 0)),
                  pl.BlockSpec((D, H), lambda i: (0, 0)),
                  pl.BlockSpec((H, D), lambda i: (0, 0))],
        out_specs=[row, row],
        out_shape=[jax.ShapeDtypeStruct((E, D), jnp.float32),
                   jax.ShapeDtypeStruct((E, D), jnp.float32)],
    )(qi, kvj, edge_attr, edge_W, We_l, edge_b2, gsel, ssel)


def _norm_body(a0_ref, a1_ref, d0_ref, d1_ref, skip_ref, h_ref, o_ref):
    denb = d0_ref[...] + d1_ref[...]
    agg = (a0_ref[...] + a1_ref[...]) / (denb + 1e-16) + skip_ref[...]
    o_ref[...] = h_ref[...] + jnp.maximum(agg, 0.0)


def _norm_update(acc0, acc1, den0, den1, skip, h):
    bn = 2000
    grid = N // bn
    row = pl.BlockSpec((bn, D), lambda i: (i, 0))
    return pl.pallas_call(
        _norm_body,
        grid=(grid,),
        in_specs=[row, row, row, row, row, row],
        out_specs=row,
        out_shape=jax.ShapeDtypeStruct((N, D), jnp.float32),
    )(acc0, acc1, den0, den1, skip, h)


def _final_body(h_ref, g_ref, b_ref, ow_ref, ob_ref, o_ref, acc_ref):
    i = pl.program_id(0)
    nblk = pl.num_programs(0)
    hb = h_ref[...]
    mu = jnp.mean(hb, axis=1, keepdims=True)
    var = jnp.mean((hb - mu) ** 2, axis=1, keepdims=True)
    hn = (hb - mu) / jnp.sqrt(var + 1e-5) * g_ref[...] + b_ref[...]
    psum = jnp.sum(hn, axis=0, keepdims=True)

    @pl.when(i == 0)
    def _():
        acc_ref[...] = jnp.zeros_like(acc_ref)

    acc_ref[...] += psum

    @pl.when(i == nblk - 1)
    def _():
        o_ref[...] = jnp.dot(acc_ref[...] * (1.0 / N), ow_ref[...],
                             preferred_element_type=jnp.float32) + ob_ref[...]


def _final(h, ln_g2d, ln_b2d, out_W, out_b2d):
    bn = 2000
    grid = N // bn
    return pl.pallas_call(
        _final_body,
        grid=(grid,),
        in_specs=[pl.BlockSpec((bn, D), lambda i: (i, 0)),
                  pl.BlockSpec((1, D), lambda i: (0, 0)),
                  pl.BlockSpec((1, D), lambda i: (0, 0)),
                  pl.BlockSpec((D, D), lambda i: (0, 0)),
                  pl.BlockSpec((1, D), lambda i: (0, 0))],
        out_specs=pl.BlockSpec((1, D), lambda i: (0, 0)),
        out_shape=jax.ShapeDtypeStruct((1, D), jnp.float32),
        scratch_shapes=[pltpu.VMEM((1, D), jnp.float32)],
    )(h, ln_g2d, ln_b2d, out_W, out_b2d)


def _sc_gather_body(kvT, qT, src, dst, kvj_out, qi_out,
                    idxs, idxd, kvb, qb, sg0, sg1, sw0, sw1):
    cid = lax.axis_index("c")
    sid = lax.axis_index("s")
    wid = cid * NS + sid
    base = wid * EPW
    sg = (sg0, sg1)
    sw = (sw0, sw1)

    def issue_gather(c, p):
        eb = base + c * B
        pltpu.sync_copy(src.at[pl.ds(eb, B)], idxs.at[p])
        pltpu.sync_copy(dst.at[pl.ds(eb, B)], idxd.at[p])
        pltpu.async_copy(kvT.at[idxs.at[p]], kvb.at[p], sg[p])
        pltpu.async_copy(qT.at[idxd.at[p]], qb.at[p], sg[p])

    def wait_gather(p):
        pltpu.make_async_copy(kvT.at[idxs.at[p]], kvb.at[p], sg[p]).wait()
        pltpu.make_async_copy(qT.at[idxd.at[p]], qb.at[p], sg[p]).wait()

    def issue_write(c, p):
        eb = base + c * B
        pltpu.async_copy(kvb.at[p], kvj_out.at[pl.ds(eb, B)], sw[p])
        pltpu.async_copy(qb.at[p], qi_out.at[pl.ds(eb, B)], sw[p])

    def wait_write(p):
        pltpu.make_async_copy(kvb.at[p], kvj_out.at[pl.ds(base, B)], sw[p]).wait()
        pltpu.make_async_copy(qb.at[p], qi_out.at[pl.ds(base, B)], sw[p]).wait()

    issue_gather(0, 0)

    @pl.loop(0, NCH - 1, step=2)
    def _(c):
        @pl.when(c > 0)
        def _():
            wait_write(1)

        issue_gather(c + 1, 1)
        wait_gather(0)
        issue_write(c, 0)
        wait_gather(1)
        issue_write(c + 1, 1)
        wait_write(0)
        issue_gather(c + 2, 0)

    wait_gather(0)
    wait_write(1)
    issue_write(NCH - 1, 0)
    wait_write(0)


def _sc_gather(kvT, qT, src, dst):
    mesh = plsc.VectorSubcoreMesh(core_axis_name="c", subcore_axis_name="s")
    f = pl.kernel(
        _sc_gather_body,
        out_type=[jax.ShapeDtypeStruct((E, D2), jnp.float32),
                  jax.ShapeDtypeStruct((E, D), jnp.float32)],
        mesh=mesh,
        scratch_types=[
            pltpu.VMEM((2, B), jnp.int32),
            pltpu.VMEM((2, B), jnp.int32),
            pltpu.VMEM((2, B, D2), jnp.float32),
            pltpu.VMEM((2, B, D), jnp.float32),
            pltpu.SemaphoreType.DMA,
            pltpu.SemaphoreType.DMA,
            pltpu.SemaphoreType.DMA,
            pltpu.SemaphoreType.DMA,
        ],
    )
    return f(kvT, qT, src, dst)


def _sc_scatter_body(msg, dst, znode, acc_out,
                     tab_sh, idx, mb, sl0, sl1):
    cid = lax.axis_index("c")
    sid = lax.axis_index("s")
    wid = cid * NS + sid
    base = wid * EPW
    r0 = sid * RPT
    off = cid * N_PAD + r0
    sl = (sl0, sl1)

    def accumulate(src_arr, out_arr):
        pltpu.sync_copy(znode.at[pl.ds(r0, RPT)], tab_sh.at[pl.ds(r0, RPT)])
        plsc.subcore_barrier()

        def issue_load(c, p):
            eb = base + c * B
            pltpu.async_copy(dst.at[pl.ds(eb, B)], idx.at[p], sl[p])
            pltpu.async_copy(src_arr.at[pl.ds(eb, B)], mb.at[p], sl[p])

        def wait_load(p):
            pltpu.make_async_copy(dst.at[pl.ds(base, B)], idx.at[p], sl[p]).wait()
            pltpu.make_async_copy(src_arr.at[pl.ds(base, B)], mb.at[p], sl[p]).wait()

        def scatter(p):
            pltpu.sync_copy(mb.at[p], tab_sh.at[idx.at[p]], add=True)

        issue_load(0, 0)

        @pl.loop(0, NCH - 1, step=2)
        def _(c):
            issue_load(c + 1, 1)
            wait_load(0)
            scatter(0)
            issue_load(c + 2, 0)
            wait_load(1)
            scatter(1)

        wait_load(0)
        scatter(0)
        plsc.subcore_barrier()
        pltpu.sync_copy(tab_sh.at[pl.ds(r0, RPT)], out_arr.at[pl.ds(off, RPT)])
        plsc.subcore_barrier()

    accumulate(msg, acc_out)


def _sc_scatter(msg, dst, znode):
    mesh = plsc.VectorSubcoreMesh(core_axis_name="c", subcore_axis_name="s")
    f = pl.kernel(
        _sc_scatter_body,
        out_type=jax.ShapeDtypeStruct((NC * N_PAD, D), jnp.float32),
        mesh=mesh,
        scratch_types=[
            pltpu.VMEM_SHARED((N_PAD, D), jnp.float32),
            pltpu.VMEM((2, B), jnp.int32),
            pltpu.VMEM((2, B, D), jnp.float32),
            pltpu.SemaphoreType.DMA,
            pltpu.SemaphoreType.DMA,
        ],
    )
    accP = f(msg, dst, znode)
    return accP.reshape(NC, N_PAD, D)


def kernel(x, edge_index, edge_attr, node_W, node_b, edge_W, edge_b, Wq, bq,
           Wk, bk, Wv, bv, We, Wskip, bskip, ln_g, ln_b, out_W, out_b):
    src = edge_index[0]
    dst = edge_index[1]

    lane = jnp.arange(D, dtype=jnp.int32)
    head = jnp.arange(H, dtype=jnp.int32)
    gsel = (lane[:, None] // C == head[None, :]).astype(jnp.float32)
    ssel = gsel.T.copy()

    znode = jnp.zeros((N_PAD, D), jnp.float32)

    node_b2 = node_b.reshape(1, D)
    edge_b2 = edge_b.reshape(1, D)

    h = _proj(x, node_W, node_b2)

    for l in range(2):
        q, kv, skip = _qkvs(h, Wq[l], Wk[l], Wv[l], Wskip[l],
                            bq[l].reshape(1, D), bk[l].reshape(1, D),
                            bv[l].reshape(1, D), bskip[l].reshape(1, D))
        kvj, qi = _sc_gather(kv, q, src, dst)
        msg, w8 = _edge_compute(qi, kvj, edge_attr, edge_W, We[l], edge_b2,
                                gsel, ssel)
        accP = _sc_scatter(msg, dst, znode)
        denP = _sc_scatter(w8, dst, znode)
        h = _norm_update(accP[0, :N], accP[1, :N], denP[0, :N], denP[1, :N],
                         skip, h)

    return _final(h, ln_g.reshape(1, D), ln_b.reshape(1, D), out_W,
                  out_b.reshape(1, D))
